# Initial kernel scaffold; baseline (speedup 1.0000x reference)
import jax
import jax.numpy as jnp
from jax.experimental import pallas as pl


def kernel(positions, cell, numbers):
    raise NotImplementedError("write your pallas kernel here")


# speedup vs baseline: 11.8763x; 11.8763x over previous
import jax
import jax.numpy as jnp
from jax.experimental import pallas as pl

_K = 9
_N_TARGET = 4000


def _knn_body(rows_per_blk, n):
    def body(xr_ref, xc_ref, idx_ref, dst_ref):
        i = pl.program_id(0)
        xr = xr_ref[...]
        xc = xc_ref[...]
        d0 = xr[:, 0:1] - xc[0:1, :]
        acc = d0 * d0
        d1 = xr[:, 1:2] - xc[1:2, :]
        acc = acc + d1 * d1
        d2 = xr[:, 2:3] - xc[2:3, :]
        acc = acc + d2 * d2
        cols = jax.lax.broadcasted_iota(jnp.int32, (rows_per_blk, n), 1)
        rows = jax.lax.broadcasted_iota(jnp.int32, (rows_per_blk, n), 0)
        rows = rows + i * rows_per_blk
        acc = jnp.where(cols == rows, acc + jnp.float32(1e9), acc)
        for k in range(_K):
            m = jnp.min(acc, axis=1, keepdims=True)
            hit = acc == m
            idx = jnp.min(jnp.where(hit, cols, n), axis=1, keepdims=True)
            idx_ref[:, k:k + 1] = idx
            dst_ref[:, k:k + 1] = jnp.sqrt(jnp.maximum(m, jnp.float32(1e-12)))
            acc = jnp.where(cols == idx, jnp.float32(jnp.inf), acc)
    return body


def _knn_graph(x):
    n = x.shape[0]
    rows_per_blk = 432
    grid = n // rows_per_blk
    xc = x.T
    idx, dists = pl.pallas_call(
        _knn_body(rows_per_blk, n),
        grid=(grid,),
        in_specs=[
            pl.BlockSpec((rows_per_blk, 3), lambda i: (i, 0)),
            pl.BlockSpec((3, n), lambda i: (0, 0)),
        ],
        out_specs=[
            pl.BlockSpec((rows_per_blk, _K), lambda i: (i, 0)),
            pl.BlockSpec((rows_per_blk, _K), lambda i: (i, 0)),
        ],
        out_shape=[
            jax.ShapeDtypeStruct((n, _K), jnp.int32),
            jax.ShapeDtypeStruct((n, _K), jnp.float32),
        ],
    )(x, xc)
    return idx, dists


def kernel(positions, cell, numbers):
    frac = positions @ jnp.linalg.inv(cell)
    replicates = int((_N_TARGET / positions.shape[0]) ** (1.0 / 3.0))
    r = replicates
    ii, jj, kk = jnp.meshgrid(jnp.arange(r), jnp.arange(r), jnp.arange(r),
                              indexing='ij')
    offs = jnp.stack([ii, jj, kk], axis=-1).reshape(-1, 3).astype(frac.dtype)
    supercell = (frac[None, :, :] + offs[:, None, :]).reshape(-1, 3)

    scale = jnp.float32(0.05)
    eps = jax.random.normal(jax.random.key(42), supercell.shape,
                            supercell.dtype)
    supercell = supercell + scale * eps

    miller = jnp.array([1.0, 1.0, 0.0], dtype=jnp.float32)
    m = miller.astype(supercell.dtype)
    msum = jnp.sum(m)
    proj = supercell @ m
    thresh = replicates * msum / 2.0
    shift = jnp.where(proj > thresh, 1.0, 0.0).astype(supercell.dtype)
    supercell = supercell - shift[:, None] * (m / jnp.maximum(msum, 1.0)) * replicates

    supercell = supercell @ cell

    src, dists = _knn_graph(supercell)
    n = supercell.shape[0]
    dst = jnp.broadcast_to(jnp.arange(n)[:, None], (n, _K))
    z = jnp.tile(numbers, r ** 3)
    return src, dst, dists, z, scale
